# Initial kernel scaffold; baseline (speedup 1.0000x reference)
import jax
import jax.numpy as jnp
from jax.experimental import pallas as pl

N_NODES = 10000
DEG = 32
D_FEAT = 128
BLOCK = 500


def _mean_kernel(x_ref, o_ref):
    o_ref[...] = jnp.sum(x_ref[...], axis=1) * (1.0 / DEG)


def kernel(neighbour):
    return pl.pallas_call(
        _mean_kernel,
        grid=(N_NODES // BLOCK,),
        in_specs=[pl.BlockSpec((BLOCK, DEG, D_FEAT), lambda i: (i, 0, 0))],
        out_specs=pl.BlockSpec((BLOCK, D_FEAT), lambda i: (i, 0)),
        out_shape=jax.ShapeDtypeStruct((N_NODES, D_FEAT), jnp.float32),
    )(neighbour)


# speedup vs baseline: 1.0821x; 1.0821x over previous
import jax
import jax.numpy as jnp
from jax.experimental import pallas as pl

N_NODES = 10000
DEG = 32
D_FEAT = 128
BLOCK = 1000


def _mean_kernel(x_ref, o_ref):
    o_ref[...] = jnp.sum(x_ref[...], axis=1) * (1.0 / DEG)


def kernel(neighbour):
    return pl.pallas_call(
        _mean_kernel,
        grid=(N_NODES // BLOCK,),
        in_specs=[pl.BlockSpec((BLOCK, DEG, D_FEAT), lambda i: (i, 0, 0))],
        out_specs=pl.BlockSpec((BLOCK, D_FEAT), lambda i: (i, 0)),
        out_shape=jax.ShapeDtypeStruct((N_NODES, D_FEAT), jnp.float32),
    )(neighbour)
